# Initial kernel scaffold; baseline (speedup 1.0000x reference)
import jax
import jax.numpy as jnp
from jax.experimental import pallas as pl


def kernel(x, lap_indices, lap_values, W1, b1, gamma, beta, W2, b2):
    raise NotImplementedError("write your pallas kernel here")


# speedup vs baseline: 2.7893x; 2.7893x over previous
import functools

import jax
import jax.numpy as jnp
from jax import lax
from jax.experimental import pallas as pl
from jax.experimental.pallas import tpu as pltpu
from jax.experimental.pallas import tpu_sc as plsc

N = 10000
E = 160000
FIN = 256
HID = 512
KORD = 4
CW = 128
NC = 2
NS = 16
EPT = E // NS
PE = 10240
PE2 = PE + 128
B = 40
NB = PE // B
NP = 10112
NPT = NP // NS
NBUF = 2

_f32 = jnp.float32
_i32 = jnp.int32


def _make_spmm(C):
    mesh = plsc.VectorSubcoreMesh(
        core_axis_name="c", subcore_axis_name="s", num_cores=NC, num_subcores=NS)
    rounds = C // NC

    @functools.partial(
        pl.kernel,
        out_type=jax.ShapeDtypeStruct((C * NP, CW), _f32),
        mesh=mesh,
        scratch_types=(
            [pltpu.VMEM((PE,), _i32)]
            + [pltpu.VMEM((PE,), _i32)]
            + [pltpu.VMEM((PE2,), _f32)]
            + [pltpu.VMEM((B, CW), _f32) for _ in range(NBUF)]
            + [pltpu.VMEM_SHARED((NP, CW), _f32)]
            + [pltpu.SemaphoreType.DMA for _ in range(2 * NBUF)]
        ),
    )
    def spmm(vflat, srcs, dsts, vals, out,
             idx_adj, dst_v, val_v,
             g0, g1, acc,
             gs0, gs1, ss0, ss1):
        c = lax.axis_index("c")
        s = lax.axis_index("s")
        gbuf = (g0, g1)
        gsem = (gs0, gs1)
        ssem = (ss0, ss1)

        pltpu.sync_copy(srcs.at[pl.ds(s * PE, PE)], idx_adj)
        pltpu.sync_copy(dsts.at[pl.ds(s * PE, PE)], dst_v)
        pltpu.sync_copy(vals.at[pl.ds(s * PE2, PE2)], val_v)

        def scale_rows(p, b):
            gp = gbuf[p]

            def row(i, _):
                v16 = val_v[pl.ds(b * B + i, 16)]
                vv = jnp.full((16,), v16[0], dtype=_f32)
                for q in range(CW // 16):
                    sl = pl.ds(q * 16, 16)
                    gp[i, sl] = gp[i, sl] * vv
                return _

            lax.fori_loop(0, B, row, None)

        def start_gather(p, b):
            pltpu.async_copy(
                vflat.at[idx_adj.at[pl.ds(b * B, B)]], gbuf[p], gsem[p])

        def wait_gather(p, b):
            pltpu.make_async_copy(
                vflat.at[idx_adj.at[pl.ds(b * B, B)]], gbuf[p], gsem[p]).wait()

        def start_scatter(p, b):
            pltpu.async_copy(
                gbuf[p], acc.at[dst_v.at[pl.ds(b * B, B)]], ssem[p], add=True)

        def wait_scatter(p, b):
            pltpu.make_async_copy(
                gbuf[p], acc.at[dst_v.at[pl.ds(b * B, B)]], ssem[p]).wait()

        for r in range(rounds):
            step = c * NP if r == 0 else NC * NP

            def zrow(i, _):
                for q in range(CW // 16):
                    g0[i, pl.ds(q * 16, 16)] = jnp.zeros((16,), _f32)
                return _
            lax.fori_loop(0, B, zrow, None)
            for q in range(NPT // B):
                pltpu.sync_copy(g0, acc.at[pl.ds(s * NPT + q * B, B)])
            rem = NPT - (NPT // B) * B
            if rem:
                pltpu.sync_copy(g0.at[pl.ds(0, rem)],
                                acc.at[pl.ds(s * NPT + (NPT // B) * B, rem)])

            def fill(m, _):
                sl = pl.ds(m * 16, 16)
                idx_adj[sl] = idx_adj[sl] + step
                return _
            lax.fori_loop(0, PE // 16, fill, None)

            plsc.subcore_barrier()

            start_gather(0, 0)
            start_gather(1, 1)
            wait_gather(0, 0)
            scale_rows(0, 0)
            start_scatter(0, 0)

            def steady(t, _):
                for k in range(NBUF):
                    b = 1 + t * NBUF + k
                    p = (1 + k) % NBUF
                    pn = (p + 1) % NBUF
                    wait_scatter(pn, b - 1)
                    start_gather(pn, b + 1)
                    wait_gather(p, b)
                    scale_rows(p, b)
                    start_scatter(p, b)
                return _
            lax.fori_loop(0, (NB - 2) // NBUF, steady, None)

            wait_scatter(0, NB - 2)
            wait_gather(1, NB - 1)
            scale_rows(1, NB - 1)
            start_scatter(1, NB - 1)
            wait_scatter(1, NB - 1)

            plsc.subcore_barrier()

            joff = (r * NC + c) * NP
            base = s * NPT
            npc = (NPT + B - 1) // B
            sizes = [B] * (npc - 1) + [NPT - (npc - 1) * B]

            def ro_in(q, start):
                ln = sizes[q]
                d = pltpu.make_async_copy(
                    acc.at[pl.ds(base + q * B, ln)],
                    gbuf[q % NBUF].at[pl.ds(0, ln)], gsem[q % NBUF])
                d.start() if start else d.wait()

            def ro_out(q, start):
                ln = sizes[q]
                d = pltpu.make_async_copy(
                    gbuf[q % NBUF].at[pl.ds(0, ln)],
                    out.at[pl.ds(joff + base + q * B, ln)], ssem[q % NBUF])
                d.start() if start else d.wait()

            ro_in(0, True)
            ro_in(1, True)
            for q in range(npc):
                ro_in(q, False)
                ro_out(q, True)
                ro_out(q, False)
                if q + 2 < npc:
                    ro_in(q + 2, True)
            if r + 1 < rounds:
                plsc.subcore_barrier()

    return spmm


_spmm2 = _make_spmm(2)
_spmm4 = _make_spmm(4)

RB = 1000


def _tc1_body(t0, t1, t2, t3, w, b, o):
    acc = jnp.zeros((RB, HID), _f32) + b[...]
    for k, tk in enumerate((t0, t1, t2, t3)):
        for c in range(2):
            acc += jnp.dot(tk[c], w[k, c], preferred_element_type=_f32)
    acc = jnp.maximum(acc, 0.0)
    for c2 in range(HID // CW):
        o[c2] = acc[:, c2 * CW:(c2 + 1) * CW]


_tc1 = pl.pallas_call(
    _tc1_body,
    grid=(N // RB,),
    in_specs=[pl.BlockSpec((2, RB, CW), lambda r: (0, r, 0))] * 4
    + [pl.BlockSpec((KORD, 2, CW, HID), lambda r: (0, 0, 0, 0)),
       pl.BlockSpec((1, HID), lambda r: (0, 0))],
    out_specs=pl.BlockSpec((HID // CW, RB, CW), lambda r: (0, r, 0)),
    out_shape=jax.ShapeDtypeStruct((HID // CW, NP, CW), _f32),
)


def _bn_body(h, g, bt, o):
    hb = h[0]
    mean = jnp.mean(hb, axis=0, keepdims=True)
    var = jnp.mean((hb - mean) ** 2, axis=0, keepdims=True)
    o[0] = g[0] * (hb - mean) / jnp.sqrt(var + 1e-5) + bt[0]


_bn = pl.pallas_call(
    _bn_body,
    grid=(HID // CW,),
    in_specs=[pl.BlockSpec((1, N, CW), lambda c: (c, 0, 0)),
              pl.BlockSpec((1, 1, CW), lambda c: (c, 0, 0)),
              pl.BlockSpec((1, 1, CW), lambda c: (c, 0, 0))],
    out_specs=pl.BlockSpec((1, N, CW), lambda c: (c, 0, 0)),
    out_shape=jax.ShapeDtypeStruct((HID // CW, NP, CW), _f32),
)


def _tc2_body(t0, t1, t2, t3, w, b, xr, o):
    acc = xr[...] + b[...]
    for k, tk in enumerate((t0, t1, t2, t3)):
        for c in range(HID // CW):
            acc += jnp.dot(tk[c], w[k, c], preferred_element_type=_f32)
    o[...] = jnp.maximum(acc, 0.0)


_tc2 = pl.pallas_call(
    _tc2_body,
    grid=(N // RB,),
    in_specs=[pl.BlockSpec((HID // CW, RB, CW), lambda r: (0, r, 0))] * 4
    + [pl.BlockSpec((KORD, HID // CW, CW, FIN), lambda r: (0, 0, 0, 0)),
       pl.BlockSpec((1, FIN), lambda r: (0, 0)),
       pl.BlockSpec((RB, FIN), lambda r: (r, 0))],
    out_specs=pl.BlockSpec((RB, FIN), lambda r: (r, 0)),
    out_shape=jax.ShapeDtypeStruct((N, FIN), _f32),
)


def _fold_weights(W):
    F = W.shape[1]
    Wt = jnp.stack([W[0] - W[2], W[1] - 3.0 * W[3], 2.0 * W[2], 4.0 * W[3]])
    return Wt.reshape(KORD, F // CW, CW, W.shape[2])


def kernel(x, lap_indices, lap_values, W1, b1, gamma, beta, W2, b2):
    src = jnp.pad(lap_indices[1].astype(_i32).reshape(NS, EPT),
                  ((0, 0), (0, PE - EPT))).reshape(NS * PE)
    dst = jnp.pad(lap_indices[0].astype(_i32).reshape(NS, EPT),
                  ((0, 0), (0, PE - EPT))).reshape(NS * PE)
    vals = jnp.pad(lap_values.astype(_f32).reshape(NS, EPT),
                   ((0, 0), (0, PE2 - EPT))).reshape(NS * PE2)

    xT = x.reshape(N, 2, CW).transpose(1, 0, 2)
    xTp = jnp.pad(xT, ((0, 0), (0, NP - N), (0, 0)))
    xTf = xTp.reshape(2 * NP, CW)

    u1 = _spmm2(xTf, src, dst, vals)
    u2 = _spmm2(u1, src, dst, vals)
    v3 = _spmm2(u2, src, dst, vals)

    h = _tc1(xTp, u1.reshape(2, NP, CW), u2.reshape(2, NP, CW),
             v3.reshape(2, NP, CW), _fold_weights(W1), b1.reshape(1, HID))
    hn = _bn(h, gamma.reshape(HID // CW, 1, CW), beta.reshape(HID // CW, 1, CW))

    hf = hn.reshape(HID // CW * NP, CW)
    w1 = _spmm4(hf, src, dst, vals)
    w2 = _spmm4(w1, src, dst, vals)
    w3 = _spmm4(w2, src, dst, vals)

    out = _tc2(hn, w1.reshape(HID // CW, NP, CW), w2.reshape(HID // CW, NP, CW),
               w3.reshape(HID // CW, NP, CW), _fold_weights(W2),
               b2.reshape(1, FIN), x)
    return out


# speedup vs baseline: 5.2684x; 1.8888x over previous
import functools

import jax
import jax.numpy as jnp
from jax import lax
from jax.experimental import pallas as pl
from jax.experimental.pallas import tpu as pltpu
from jax.experimental.pallas import tpu_sc as plsc

N = 10000
E = 160000
FIN = 256
HID = 512
KORD = 4
CW = 128
NC = 2
NS = 16
EPT = E // NS
PE = 10112
B = 32
NB = PE // B
NP = 10112
NPT = NP // NS
NBUF = 4

_f32 = jnp.float32
_i32 = jnp.int32


def _make_spmm(C):
    mesh = plsc.VectorSubcoreMesh(
        core_axis_name="c", subcore_axis_name="s", num_cores=NC, num_subcores=NS)
    rounds = C // NC

    @functools.partial(
        pl.kernel,
        out_type=jax.ShapeDtypeStruct((C * NP, CW), _f32),
        mesh=mesh,
        scratch_types=(
            [pltpu.VMEM((PE,), _i32)]
            + [pltpu.VMEM((PE,), _i32)]
            + [pltpu.VMEM((PE,), _f32)]
            + [pltpu.VMEM((B, CW), _f32) for _ in range(NBUF)]
            + [pltpu.VMEM_SHARED((NP, CW), _f32)]
            + [pltpu.SemaphoreType.DMA for _ in range(2 * NBUF)]
        ),
    )
    def spmm(vflat, srcs, dsts, vals, out,
             idx_adj, dst_v, val_v,
             g0, g1, g2, g3, acc,
             gs0, gs1, gs2, gs3, ss0, ss1, ss2, ss3):
        c = lax.axis_index("c")
        s = lax.axis_index("s")
        gbuf = (g0, g1, g2, g3)
        gsem = (gs0, gs1, gs2, gs3)
        ssem = (ss0, ss1, ss2, ss3)

        pltpu.sync_copy(srcs.at[pl.ds(s * PE, PE)], idx_adj)
        pltpu.sync_copy(dsts.at[pl.ds(s * PE, PE)], dst_v)
        pltpu.sync_copy(vals.at[pl.ds(s * PE, PE)], val_v)

        def scale_rows(p, b):
            gp = gbuf[p]

            @functools.partial(plsc.parallel_loop, 0, B, unroll=4)
            def row(i):
                st = jnp.minimum(b * B + i, PE - 16)
                v16 = val_v[pl.ds(st, 16)]
                vv = jnp.full((16,), v16[0], dtype=_f32)
                for q in range(CW // 16):
                    sl = pl.ds(q * 16, 16)
                    gp[i, sl] = gp[i, sl] * vv

        def start_gather(p, b):
            pltpu.async_copy(
                vflat.at[idx_adj.at[pl.ds(b * B, B)]], gbuf[p], gsem[p])

        def wait_gather(p, b):
            pltpu.make_async_copy(
                vflat.at[idx_adj.at[pl.ds(b * B, B)]], gbuf[p], gsem[p]).wait()

        def start_scatter(p, b):
            pltpu.async_copy(
                gbuf[p], acc.at[dst_v.at[pl.ds(b * B, B)]], ssem[p], add=True)

        def wait_scatter(p, b):
            pltpu.make_async_copy(
                gbuf[p], acc.at[dst_v.at[pl.ds(b * B, B)]], ssem[p]).wait()

        for r in range(rounds):
            step = c * NP if r == 0 else NC * NP

            def zrow(i, _):
                for q in range(CW // 16):
                    g0[i, pl.ds(q * 16, 16)] = jnp.zeros((16,), _f32)
                return _
            lax.fori_loop(0, B, zrow, None)
            for q in range(NPT // B):
                pltpu.sync_copy(g0, acc.at[pl.ds(s * NPT + q * B, B)])
            rem = NPT - (NPT // B) * B
            if rem:
                pltpu.sync_copy(g0.at[pl.ds(0, rem)],
                                acc.at[pl.ds(s * NPT + (NPT // B) * B, rem)])

            def fill(m, _):
                sl = pl.ds(m * 16, 16)
                idx_adj[sl] = idx_adj[sl] + step
                return _
            lax.fori_loop(0, PE // 16, fill, None)

            plsc.subcore_barrier()

            start_gather(0, 0)
            start_gather(1, 1)
            for b in (0, 1):
                start_gather(b + 2, b + 2)
                wait_gather(b, b)
                scale_rows(b, b)
                start_scatter(b, b)

            def steady(t, _):
                for k in range(NBUF):
                    b = 2 + t * NBUF + k
                    p = (2 + k) % NBUF
                    p2 = (p + 2) % NBUF
                    wait_scatter(p2, b - 2)
                    start_gather(p2, b + 2)
                    wait_gather(p, b)
                    scale_rows(p, b)
                    start_scatter(p, b)
                return _
            lax.fori_loop(0, (NB - 4) // NBUF, steady, None)

            for b in (NB - 2, NB - 1):
                p = b % NBUF
                p2 = (p + 2) % NBUF
                wait_scatter(p2, b - 2)
                wait_gather(p, b)
                scale_rows(p, b)
                start_scatter(p, b)
            wait_scatter((NB - 2) % NBUF, NB - 2)
            wait_scatter((NB - 1) % NBUF, NB - 1)

            plsc.subcore_barrier()

            joff = (r * NC + c) * NP
            base = s * NPT
            npc = (NPT + B - 1) // B
            sizes = [B] * (npc - 1) + [NPT - (npc - 1) * B]

            def ro_in(q, start):
                ln = sizes[q]
                d = pltpu.make_async_copy(
                    acc.at[pl.ds(base + q * B, ln)],
                    gbuf[q % NBUF].at[pl.ds(0, ln)], gsem[q % NBUF])
                d.start() if start else d.wait()

            def ro_out(q, start):
                ln = sizes[q]
                d = pltpu.make_async_copy(
                    gbuf[q % NBUF].at[pl.ds(0, ln)],
                    out.at[pl.ds(joff + base + q * B, ln)], ssem[q % NBUF])
                d.start() if start else d.wait()

            ro_in(0, True)
            ro_in(1, True)
            for q in range(npc):
                ro_in(q, False)
                ro_out(q, True)
                if q + 2 < npc:
                    if q >= 2:
                        ro_out(q - 2, False)
                    ro_in(q + 2, True)
            for q in range(max(0, npc - 4), npc):
                ro_out(q, False)
            if r + 1 < rounds:
                plsc.subcore_barrier()

    return spmm


_spmm2 = _make_spmm(2)
_spmm4 = _make_spmm(4)

RB = 1000


def _tc1_body(t0, t1, t2, t3, w, b, o):
    acc = jnp.zeros((RB, HID), _f32) + b[...]
    for k, tk in enumerate((t0, t1, t2, t3)):
        for c in range(2):
            acc += jnp.dot(tk[c], w[k, c], preferred_element_type=_f32)
    acc = jnp.maximum(acc, 0.0)
    for c2 in range(HID // CW):
        o[c2] = acc[:, c2 * CW:(c2 + 1) * CW]


_tc1 = pl.pallas_call(
    _tc1_body,
    grid=(N // RB,),
    in_specs=[pl.BlockSpec((2, RB, CW), lambda r: (0, r, 0))] * 4
    + [pl.BlockSpec((KORD, 2, CW, HID), lambda r: (0, 0, 0, 0)),
       pl.BlockSpec((1, HID), lambda r: (0, 0))],
    out_specs=pl.BlockSpec((HID // CW, RB, CW), lambda r: (0, r, 0)),
    out_shape=jax.ShapeDtypeStruct((HID // CW, NP, CW), _f32),
)


def _bn_body(h, g, bt, o):
    hb = h[0]
    mean = jnp.mean(hb, axis=0, keepdims=True)
    var = jnp.mean((hb - mean) ** 2, axis=0, keepdims=True)
    o[0] = g[0] * (hb - mean) / jnp.sqrt(var + 1e-5) + bt[0]


_bn = pl.pallas_call(
    _bn_body,
    grid=(HID // CW,),
    in_specs=[pl.BlockSpec((1, N, CW), lambda c: (c, 0, 0)),
              pl.BlockSpec((1, 1, CW), lambda c: (c, 0, 0)),
              pl.BlockSpec((1, 1, CW), lambda c: (c, 0, 0))],
    out_specs=pl.BlockSpec((1, N, CW), lambda c: (c, 0, 0)),
    out_shape=jax.ShapeDtypeStruct((HID // CW, NP, CW), _f32),
)


def _tc2_body(t0, t1, t2, t3, w, b, xr, o):
    acc = xr[...] + b[...]
    for k, tk in enumerate((t0, t1, t2, t3)):
        for c in range(HID // CW):
            acc += jnp.dot(tk[c], w[k, c], preferred_element_type=_f32)
    o[...] = jnp.maximum(acc, 0.0)


_tc2 = pl.pallas_call(
    _tc2_body,
    grid=(N // RB,),
    in_specs=[pl.BlockSpec((HID // CW, RB, CW), lambda r: (0, r, 0))] * 4
    + [pl.BlockSpec((KORD, HID // CW, CW, FIN), lambda r: (0, 0, 0, 0)),
       pl.BlockSpec((1, FIN), lambda r: (0, 0)),
       pl.BlockSpec((RB, FIN), lambda r: (r, 0))],
    out_specs=pl.BlockSpec((RB, FIN), lambda r: (r, 0)),
    out_shape=jax.ShapeDtypeStruct((N, FIN), _f32),
)


def _fold_weights(W):
    F = W.shape[1]
    Wt = jnp.stack([W[0] - W[2], W[1] - 3.0 * W[3], 2.0 * W[2], 4.0 * W[3]])
    return Wt.reshape(KORD, F // CW, CW, W.shape[2])


def kernel(x, lap_indices, lap_values, W1, b1, gamma, beta, W2, b2):
    src = jnp.pad(lap_indices[1].astype(_i32).reshape(NS, EPT),
                  ((0, 0), (0, PE - EPT))).reshape(NS * PE)
    dst = jnp.pad(lap_indices[0].astype(_i32).reshape(NS, EPT),
                  ((0, 0), (0, PE - EPT))).reshape(NS * PE)
    vals = jnp.pad(lap_values.astype(_f32).reshape(NS, EPT),
                   ((0, 0), (0, PE - EPT))).reshape(NS * PE)

    xT = x.reshape(N, 2, CW).transpose(1, 0, 2)
    xTp = jnp.pad(xT, ((0, 0), (0, NP - N), (0, 0)))
    xTf = xTp.reshape(2 * NP, CW)

    u1 = _spmm2(xTf, src, dst, vals)
    u2 = _spmm2(u1, src, dst, vals)
    v3 = _spmm2(u2, src, dst, vals)

    h = _tc1(xTp, u1.reshape(2, NP, CW), u2.reshape(2, NP, CW),
             v3.reshape(2, NP, CW), _fold_weights(W1), b1.reshape(1, HID))
    hn = _bn(h, gamma.reshape(HID // CW, 1, CW), beta.reshape(HID // CW, 1, CW))

    hf = hn.reshape(HID // CW * NP, CW)
    w1 = _spmm4(hf, src, dst, vals)
    w2 = _spmm4(w1, src, dst, vals)
    w3 = _spmm4(w2, src, dst, vals)

    out = _tc2(hn, w1.reshape(HID // CW, NP, CW), w2.reshape(HID // CW, NP, CW),
               w3.reshape(HID // CW, NP, CW), _fold_weights(W2),
               b2.reshape(1, FIN), x)
    return out
